# baseline (speedup 1.0000x reference)
import jax
import jax.numpy as jnp
from jax import lax
from jax.experimental import pallas as pl
from jax.experimental.pallas import tpu as pltpu
from jax.experimental.pallas import tpu_sc as plsc

NUM_CORES = 2
NUM_SUBCORES = 16
NUM_WORKERS = NUM_CORES * NUM_SUBCORES
CHUNK = 1280
LANES = 16


def _gather_flat(embed_lookup, flat_idx, num_indices, dim):
    per_worker = num_indices // NUM_WORKERS
    num_chunks = per_worker // CHUNK
    half = num_chunks // 2

    mesh = plsc.VectorSubcoreMesh(core_axis_name="c", subcore_axis_name="s")

    @pl.kernel(
        out_type=jax.ShapeDtypeStruct((num_indices, dim), embed_lookup.dtype),
        mesh=mesh,
        compiler_params=pltpu.CompilerParams(use_tc_tiling_on_sc=False),
        scratch_types=[
            pltpu.VMEM((CHUNK,), jnp.int32),
            pltpu.VMEM((CHUNK,), jnp.int32),
            pltpu.VMEM((CHUNK, dim), jnp.float32),
            pltpu.VMEM((CHUNK, dim), jnp.float32),
            pltpu.SemaphoreType.DMA,
            pltpu.SemaphoreType.DMA,
            pltpu.SemaphoreType.DMA,
            pltpu.SemaphoreType.DMA,
        ],
    )
    def gather_kernel(tab_hbm, idx_hbm, out_hbm, idx0, idx1, rows0, rows1,
                      g0, g1, s0, s1):
        wid = lax.axis_index("s") * NUM_CORES + lax.axis_index("c")
        wbase = wid * per_worker

        def load_idx(chunk, idx_v):
            pltpu.sync_copy(
                idx_hbm.at[pl.ds(wbase + chunk * CHUNK, CHUNK)], idx_v
            )

        def gather_desc(idx_v, rows_v, sem):
            return pltpu.make_async_copy(tab_hbm.at[idx_v], rows_v, sem)

        def store_desc(chunk, rows_v, sem):
            return pltpu.make_async_copy(
                rows_v, out_hbm.at[pl.ds(wbase + chunk * CHUNK, CHUNK)], sem
            )

        load_idx(0, idx0)
        gather_desc(idx0, rows0, g0).start()
        load_idx(1, idx1)
        gather_desc(idx1, rows1, g1).start()

        @pl.loop(0, half)
        def _(k):
            gather_desc(idx0, rows0, g0).wait()
            store_desc(2 * k, rows0, s0).start()
            gather_desc(idx1, rows1, g1).wait()
            store_desc(2 * k + 1, rows1, s1).start()

            @pl.when(k < half - 1)
            def _():
                load_idx(2 * k + 2, idx0)
                store_desc(2 * k, rows0, s0).wait()
                gather_desc(idx0, rows0, g0).start()
                load_idx(2 * k + 3, idx1)
                store_desc(2 * k + 1, rows1, s1).wait()
                gather_desc(idx1, rows1, g1).start()

        store_desc(num_chunks - 2, rows0, s0).wait()
        store_desc(num_chunks - 1, rows1, s1).wait()

    return gather_kernel(embed_lookup, flat_idx)


def _relayout(flat1d, hist, dim, batch):
    blk = 128
    n_units = hist * (batch // blk)
    per_worker = n_units // NUM_WORKERS
    half = per_worker // 2
    chunk_elems = blk * dim

    mesh = plsc.VectorSubcoreMesh(core_axis_name="c", subcore_axis_name="s")

    @pl.kernel(
        out_type=jax.ShapeDtypeStruct((hist, dim, batch), jnp.float32),
        mesh=mesh,
        compiler_params=pltpu.CompilerParams(needs_layout_passes=False),
        scratch_types=[
            pltpu.VMEM((chunk_elems,), jnp.float32),
            pltpu.VMEM((chunk_elems,), jnp.float32),
            pltpu.VMEM((dim, blk + 5), jnp.float32),
            pltpu.VMEM((dim, blk + 5), jnp.float32),
            pltpu.SemaphoreType.DMA,
            pltpu.SemaphoreType.DMA,
            pltpu.SemaphoreType.DMA,
            pltpu.SemaphoreType.DMA,
        ],
    )
    def relayout_kernel(in_hbm, out_hbm, x0, x1, o0, o1, l0, l1, s0, s1):
        wid = lax.axis_index("s") * NUM_CORES + lax.axis_index("c")
        ubase = wid * per_worker

        def unit_hb(u):
            return u // (batch // blk), u % (batch // blk)

        def load_desc(u, x_v, sem):
            h, b = unit_hb(u)
            base = (h * batch + b * blk) * dim
            return pltpu.make_async_copy(
                in_hbm.at[pl.ds(base, chunk_elems)], x_v, sem
            )

        def store_desc(u, o_v, sem):
            h, b = unit_hb(u)
            return pltpu.make_async_copy(
                o_v.at[:, pl.ds(0, blk)],
                out_hbm.at[h, :, pl.ds(b * blk, blk)],
                sem,
            )

        def transpose(x_v, o_v):
            iota = lax.iota(jnp.int32, LANES)
            zero = iota * 0

            @pl.loop(0, blk, step=8)
            def _(c):
                for cc in range(8):
                    idx_c = zero + (c + cc)
                    for d0 in range(0, dim, LANES):
                        vec = x_v[pl.ds((c + cc) * dim + d0, LANES)]
                        plsc.store_scatter(o_v, [iota + d0, idx_c], vec)

        load_desc(ubase, x0, l0).start()
        load_desc(ubase + 1, x1, l1).start()

        @pl.loop(0, half)
        def _(k):
            ua = ubase + 2 * k
            ub = ua + 1

            load_desc(ua, x0, l0).wait()

            @pl.when(k > 0)
            def _():
                store_desc(ua - 2, o0, s0).wait()

            transpose(x0, o0)
            store_desc(ua, o0, s0).start()

            @pl.when(k < half - 1)
            def _():
                load_desc(ua + 2, x0, l0).start()

            load_desc(ub, x1, l1).wait()

            @pl.when(k > 0)
            def _():
                store_desc(ub - 2, o1, s1).wait()

            transpose(x1, o1)
            store_desc(ub, o1, s1).start()

            @pl.when(k < half - 1)
            def _():
                load_desc(ub + 2, x1, l1).start()

        store_desc(ubase + per_worker - 2, o0, s0).wait()
        store_desc(ubase + per_worker - 1, o1, s1).wait()

    return relayout_kernel(flat1d)


def kernel(inputs, embed_lookup):
    batch, hist = inputs.shape
    dim = embed_lookup.shape[1]
    num_indices = batch * hist

    flat_idx = jnp.transpose(inputs).reshape(num_indices)

    flat = _gather_flat(embed_lookup, flat_idx, num_indices, dim)
    o_t = _relayout(flat.reshape(num_indices * dim), hist, dim, batch)
    return jnp.transpose(o_t, (2, 0, 1))


# speedup vs baseline: 1.1542x; 1.1542x over previous
import jax
import jax.numpy as jnp
from jax import lax
from jax.experimental import pallas as pl
from jax.experimental.pallas import tpu as pltpu
from jax.experimental.pallas import tpu_sc as plsc

NUM_CORES = 2
NUM_SUBCORES = 16
NUM_WORKERS = NUM_CORES * NUM_SUBCORES
CHUNK = 1280
LANES = 16


def _gather_flat(embed_lookup, flat_idx, num_indices, dim):
    per_worker = num_indices // NUM_WORKERS
    num_chunks = per_worker // CHUNK
    half = num_chunks // 2

    mesh = plsc.VectorSubcoreMesh(core_axis_name="c", subcore_axis_name="s")

    @pl.kernel(
        out_type=jax.ShapeDtypeStruct((num_indices, dim), embed_lookup.dtype),
        mesh=mesh,
        compiler_params=pltpu.CompilerParams(use_tc_tiling_on_sc=False),
        scratch_types=[
            pltpu.VMEM((CHUNK,), jnp.int32),
            pltpu.VMEM((CHUNK,), jnp.int32),
            pltpu.VMEM((CHUNK, dim), jnp.float32),
            pltpu.VMEM((CHUNK, dim), jnp.float32),
            pltpu.SemaphoreType.DMA,
            pltpu.SemaphoreType.DMA,
            pltpu.SemaphoreType.DMA,
            pltpu.SemaphoreType.DMA,
        ],
    )
    def gather_kernel(tab_hbm, idx_hbm, out_hbm, idx0, idx1, rows0, rows1,
                      g0, g1, s0, s1):
        wid = lax.axis_index("s") * NUM_CORES + lax.axis_index("c")
        wbase = wid * per_worker

        def load_idx(chunk, idx_v):
            pltpu.sync_copy(
                idx_hbm.at[pl.ds(wbase + chunk * CHUNK, CHUNK)], idx_v
            )

        def gather_desc(idx_v, rows_v, sem):
            return pltpu.make_async_copy(tab_hbm.at[idx_v], rows_v, sem)

        def store_desc(chunk, rows_v, sem):
            return pltpu.make_async_copy(
                rows_v, out_hbm.at[pl.ds(wbase + chunk * CHUNK, CHUNK)], sem
            )

        load_idx(0, idx0)
        gather_desc(idx0, rows0, g0).start()
        load_idx(1, idx1)
        gather_desc(idx1, rows1, g1).start()

        @pl.loop(0, half)
        def _(k):
            gather_desc(idx0, rows0, g0).wait()
            store_desc(2 * k, rows0, s0).start()
            gather_desc(idx1, rows1, g1).wait()
            store_desc(2 * k + 1, rows1, s1).start()

            @pl.when(k < half - 1)
            def _():
                load_idx(2 * k + 2, idx0)
                store_desc(2 * k, rows0, s0).wait()
                gather_desc(idx0, rows0, g0).start()
                load_idx(2 * k + 3, idx1)
                store_desc(2 * k + 1, rows1, s1).wait()
                gather_desc(idx1, rows1, g1).start()

        store_desc(num_chunks - 2, rows0, s0).wait()
        store_desc(num_chunks - 1, rows1, s1).wait()

    return gather_kernel(embed_lookup, flat_idx)


def _relayout(flat1d, hist, dim, batch):
    blk = 128
    n_units = hist * (batch // blk)
    per_worker = n_units // NUM_WORKERS
    half = per_worker // 2
    chunk_elems = blk * dim

    mesh = plsc.VectorSubcoreMesh(core_axis_name="c", subcore_axis_name="s")

    @pl.kernel(
        out_type=jax.ShapeDtypeStruct((hist, dim, batch), jnp.float32),
        mesh=mesh,
        compiler_params=pltpu.CompilerParams(needs_layout_passes=False),
        scratch_types=[
            pltpu.VMEM((chunk_elems,), jnp.float32),
            pltpu.VMEM((chunk_elems,), jnp.float32),
            pltpu.VMEM((dim, blk + 5), jnp.float32),
            pltpu.VMEM((dim, blk + 5), jnp.float32),
            pltpu.SemaphoreType.DMA,
            pltpu.SemaphoreType.DMA,
            pltpu.SemaphoreType.DMA,
            pltpu.SemaphoreType.DMA,
        ],
    )
    def relayout_kernel(in_hbm, out_hbm, x0, x1, o0, o1, l0, l1, s0, s1):
        wid = lax.axis_index("s") * NUM_CORES + lax.axis_index("c")
        ubase = wid * per_worker

        def unit_hb(u):
            return u // (batch // blk), u % (batch // blk)

        def load_desc(u, x_v, sem):
            h, b = unit_hb(u)
            base = (h * batch + b * blk) * dim
            return pltpu.make_async_copy(
                in_hbm.at[pl.ds(base, chunk_elems)], x_v, sem
            )

        def store_desc(u, o_v, sem):
            h, b = unit_hb(u)
            return pltpu.make_async_copy(
                o_v.at[:, pl.ds(0, blk)],
                out_hbm.at[h, :, pl.ds(b * blk, blk)],
                sem,
            )

        def transpose(x_v, o_v):
            iota = lax.iota(jnp.int32, LANES)
            zero = iota * 0

            @plsc.parallel_loop(0, blk, unroll=8)
            def _(c):
                idx_c = zero + c
                for d0 in range(0, dim, LANES):
                    vec = x_v[pl.ds(c * dim + d0, LANES)]
                    plsc.store_scatter(o_v, [iota + d0, idx_c], vec)

        load_desc(ubase, x0, l0).start()
        load_desc(ubase + 1, x1, l1).start()

        @pl.loop(0, half)
        def _(k):
            ua = ubase + 2 * k
            ub = ua + 1

            load_desc(ua, x0, l0).wait()

            @pl.when(k > 0)
            def _():
                store_desc(ua - 2, o0, s0).wait()

            transpose(x0, o0)
            store_desc(ua, o0, s0).start()

            @pl.when(k < half - 1)
            def _():
                load_desc(ua + 2, x0, l0).start()

            load_desc(ub, x1, l1).wait()

            @pl.when(k > 0)
            def _():
                store_desc(ub - 2, o1, s1).wait()

            transpose(x1, o1)
            store_desc(ub, o1, s1).start()

            @pl.when(k < half - 1)
            def _():
                load_desc(ub + 2, x1, l1).start()

        store_desc(ubase + per_worker - 2, o0, s0).wait()
        store_desc(ubase + per_worker - 1, o1, s1).wait()

    return relayout_kernel(flat1d)


def kernel(inputs, embed_lookup):
    batch, hist = inputs.shape
    dim = embed_lookup.shape[1]
    num_indices = batch * hist

    flat_idx = jnp.transpose(inputs).reshape(num_indices)

    flat = _gather_flat(embed_lookup, flat_idx, num_indices, dim)
    o_t = _relayout(flat.reshape(num_indices * dim), hist, dim, batch)
    return jnp.transpose(o_t, (2, 0, 1))


# speedup vs baseline: 1.1544x; 1.0001x over previous
import jax
import jax.numpy as jnp
from jax import lax
from jax.experimental import pallas as pl
from jax.experimental.pallas import tpu as pltpu
from jax.experimental.pallas import tpu_sc as plsc

NUM_CORES = 2
NUM_SUBCORES = 16
NUM_WORKERS = NUM_CORES * NUM_SUBCORES
CHUNK = 1280
LANES = 16


def _gather_flat(embed_lookup, flat_idx, num_indices, dim):
    per_worker = num_indices // NUM_WORKERS
    num_chunks = per_worker // CHUNK
    half = num_chunks // 2

    mesh = plsc.VectorSubcoreMesh(core_axis_name="c", subcore_axis_name="s")

    @pl.kernel(
        out_type=jax.ShapeDtypeStruct((num_indices, dim), embed_lookup.dtype),
        mesh=mesh,
        compiler_params=pltpu.CompilerParams(use_tc_tiling_on_sc=False),
        scratch_types=[
            pltpu.VMEM((CHUNK,), jnp.int32),
            pltpu.VMEM((CHUNK,), jnp.int32),
            pltpu.VMEM((CHUNK, dim), jnp.float32),
            pltpu.VMEM((CHUNK, dim), jnp.float32),
            pltpu.SemaphoreType.DMA,
            pltpu.SemaphoreType.DMA,
            pltpu.SemaphoreType.DMA,
            pltpu.SemaphoreType.DMA,
        ],
    )
    def gather_kernel(tab_hbm, idx_hbm, out_hbm, idx0, idx1, rows0, rows1,
                      g0, g1, s0, s1):
        wid = lax.axis_index("s") * NUM_CORES + lax.axis_index("c")
        wbase = wid * per_worker

        def load_idx(chunk, idx_v):
            pltpu.sync_copy(
                idx_hbm.at[pl.ds(wbase + chunk * CHUNK, CHUNK)], idx_v
            )

        def gather_desc(idx_v, rows_v, sem):
            return pltpu.make_async_copy(tab_hbm.at[idx_v], rows_v, sem)

        def store_desc(chunk, rows_v, sem):
            return pltpu.make_async_copy(
                rows_v, out_hbm.at[pl.ds(wbase + chunk * CHUNK, CHUNK)], sem
            )

        load_idx(0, idx0)
        gather_desc(idx0, rows0, g0).start()
        load_idx(1, idx1)
        gather_desc(idx1, rows1, g1).start()

        @pl.loop(0, half)
        def _(k):
            gather_desc(idx0, rows0, g0).wait()
            store_desc(2 * k, rows0, s0).start()
            gather_desc(idx1, rows1, g1).wait()
            store_desc(2 * k + 1, rows1, s1).start()

            @pl.when(k < half - 1)
            def _():
                load_idx(2 * k + 2, idx0)
                store_desc(2 * k, rows0, s0).wait()
                gather_desc(idx0, rows0, g0).start()
                load_idx(2 * k + 3, idx1)
                store_desc(2 * k + 1, rows1, s1).wait()
                gather_desc(idx1, rows1, g1).start()

        store_desc(num_chunks - 2, rows0, s0).wait()
        store_desc(num_chunks - 1, rows1, s1).wait()

    return gather_kernel(embed_lookup, flat_idx)


def _relayout(flat1d, hist, dim, batch):
    blk = 128
    n_units = hist * (batch // blk)
    per_worker = n_units // NUM_WORKERS
    half = per_worker // 2
    chunk_elems = blk * dim

    mesh = plsc.VectorSubcoreMesh(core_axis_name="c", subcore_axis_name="s")

    @pl.kernel(
        out_type=jax.ShapeDtypeStruct((hist, dim, batch), jnp.float32),
        mesh=mesh,
        compiler_params=pltpu.CompilerParams(needs_layout_passes=False),
        scratch_types=[
            pltpu.VMEM((chunk_elems,), jnp.float32),
            pltpu.VMEM((chunk_elems,), jnp.float32),
            pltpu.VMEM((dim, blk + 5), jnp.float32),
            pltpu.VMEM((dim, blk + 5), jnp.float32),
            pltpu.SemaphoreType.DMA,
            pltpu.SemaphoreType.DMA,
            pltpu.SemaphoreType.DMA,
            pltpu.SemaphoreType.DMA,
        ],
    )
    def relayout_kernel(in_hbm, out_hbm, x0, x1, o0, o1, l0, l1, s0, s1):
        wid = lax.axis_index("s") * NUM_CORES + lax.axis_index("c")
        ubase = wid * per_worker

        def unit_hb(u):
            return u // (batch // blk), u % (batch // blk)

        def load_desc(u, x_v, sem):
            h, b = unit_hb(u)
            base = (h * batch + b * blk) * dim
            return pltpu.make_async_copy(
                in_hbm.at[pl.ds(base, chunk_elems)], x_v, sem
            )

        def store_desc(u, o_v, sem):
            h, b = unit_hb(u)
            return pltpu.make_async_copy(
                o_v.at[:, pl.ds(0, blk)],
                out_hbm.at[h, :, pl.ds(b * blk, blk)],
                sem,
            )

        def transpose(x_v, o_v):
            iota = lax.iota(jnp.int32, LANES)
            zero = iota * 0

            @plsc.parallel_loop(0, blk, unroll=16)
            def _(c):
                idx_c = zero + c
                for d0 in range(0, dim, LANES):
                    vec = x_v[pl.ds(c * dim + d0, LANES)]
                    plsc.store_scatter(o_v, [iota + d0, idx_c], vec)

        load_desc(ubase, x0, l0).start()
        load_desc(ubase + 1, x1, l1).start()

        @pl.loop(0, half)
        def _(k):
            ua = ubase + 2 * k
            ub = ua + 1

            load_desc(ua, x0, l0).wait()

            @pl.when(k > 0)
            def _():
                store_desc(ua - 2, o0, s0).wait()

            transpose(x0, o0)
            store_desc(ua, o0, s0).start()

            @pl.when(k < half - 1)
            def _():
                load_desc(ua + 2, x0, l0).start()

            load_desc(ub, x1, l1).wait()

            @pl.when(k > 0)
            def _():
                store_desc(ub - 2, o1, s1).wait()

            transpose(x1, o1)
            store_desc(ub, o1, s1).start()

            @pl.when(k < half - 1)
            def _():
                load_desc(ub + 2, x1, l1).start()

        store_desc(ubase + per_worker - 2, o0, s0).wait()
        store_desc(ubase + per_worker - 1, o1, s1).wait()

    return relayout_kernel(flat1d)


def kernel(inputs, embed_lookup):
    batch, hist = inputs.shape
    dim = embed_lookup.shape[1]
    num_indices = batch * hist

    flat_idx = jnp.transpose(inputs).reshape(num_indices)

    flat = _gather_flat(embed_lookup, flat_idx, num_indices, dim)
    o_t = _relayout(flat.reshape(num_indices * dim), hist, dim, batch)
    return jnp.transpose(o_t, (2, 0, 1))
